# baseline (speedup 1.0000x reference)
import functools

import jax
import jax.numpy as jnp
from jax import lax
from jax.experimental import pallas as pl
from jax.experimental.pallas import tpu as pltpu
from jax.experimental.pallas import tpu_sc as plsc

B = 16384
D = 32

_NC, _NS = 2, 16
_NW = _NC * _NS
_BPW = B // _NW


def _sc_gather_body(user_h, item_h, t_mfu, t_mfi, t_mlu, t_mli,
                    o_mfu, o_mfi, o_mlu, o_mli,
                    uidx, iidx, r0, r1, r2, r3, sem):
    wid = lax.axis_index("s") * _NC + lax.axis_index("c")
    base = wid * _BPW
    pltpu.sync_copy(user_h.at[pl.ds(base, _BPW)], uidx)
    pltpu.sync_copy(item_h.at[pl.ds(base, _BPW)], iidx)
    c0 = pltpu.async_copy(t_mfu.at[uidx], r0, sem)
    c1 = pltpu.async_copy(t_mfi.at[iidx], r1, sem)
    c2 = pltpu.async_copy(t_mlu.at[uidx], r2, sem)
    c3 = pltpu.async_copy(t_mli.at[iidx], r3, sem)
    c0.wait()
    pltpu.sync_copy(r0, o_mfu.at[pl.ds(base, _BPW)])
    c1.wait()
    pltpu.sync_copy(r1, o_mfi.at[pl.ds(base, _BPW)])
    c2.wait()
    pltpu.sync_copy(r2, o_mlu.at[pl.ds(base, _BPW)])
    c3.wait()
    pltpu.sync_copy(r3, o_mli.at[pl.ds(base, _BPW)])


@functools.cache
def _sc_gather():
    return pl.kernel(
        _sc_gather_body,
        mesh=plsc.VectorSubcoreMesh(core_axis_name="c", subcore_axis_name="s"),
        compiler_params=pltpu.CompilerParams(use_tc_tiling_on_sc=False),
        out_type=[jax.ShapeDtypeStruct((B, D), jnp.float32)] * 4,
        scratch_types=[
            pltpu.VMEM((_BPW,), jnp.int32),
            pltpu.VMEM((_BPW,), jnp.int32),
            pltpu.VMEM((_BPW, D), jnp.float32),
            pltpu.VMEM((_BPW, D), jnp.float32),
            pltpu.VMEM((_BPW, D), jnp.float32),
            pltpu.VMEM((_BPW, D), jnp.float32),
            pltpu.SemaphoreType.DMA,
        ],
    )


def _tc_dense_body(mfu, mfi, mlu, mli, w1, b1, w2, b2, w3, b3,
                   wp_mf, wp_mlp, bp, out):
    mf_term = jnp.sum(mfu[...] * mfi[...] * wp_mf[...], axis=1, keepdims=True)
    x = jnp.concatenate([mlu[...], mli[...]], axis=1)
    dn = (((1,), (1,)), ((), ()))
    h = jnp.maximum(lax.dot_general(x, w1[...], dn,
                                    preferred_element_type=jnp.float32)
                    + b1[...], 0.0)
    h = jnp.maximum(lax.dot_general(h, w2[...], dn,
                                    preferred_element_type=jnp.float32)
                    + b2[...], 0.0)
    h = jnp.maximum(lax.dot_general(h, w3[...], dn,
                                    preferred_element_type=jnp.float32)
                    + b3[...], 0.0)
    mlp_term = jnp.sum(h * wp_mlp[...], axis=1, keepdims=True)
    out[...] = jax.nn.sigmoid(mf_term + mlp_term + bp[...])


def _tc_dense(mfu, mfi, mlu, mli, w1, b1, w2, b2, w3, b3, wp_mf, wp_mlp, bp):
    return pl.pallas_call(
        _tc_dense_body,
        out_shape=jax.ShapeDtypeStruct((B, 1), jnp.float32),
    )(mfu, mfi, mlu, mli, w1, b1, w2, b2, w3, b3, wp_mf, wp_mlp, bp)


def kernel(user, item, emb_MF_users, emb_MF_items, emb_MLP_users,
           emb_MLP_items, mlp1_weights, mlp1_bias, mlp2_weights, mlp2_bias,
           mlp3_weights, mlp3_bias, predict_weights, predict_bias):
    mfu, mfi, mlu, mli = _sc_gather()(
        user.astype(jnp.int32), item.astype(jnp.int32),
        emb_MF_users, emb_MF_items, emb_MLP_users, emb_MLP_items)
    out = _tc_dense(
        mfu, mfi, mlu, mli,
        mlp1_weights, mlp1_bias.reshape(1, -1),
        mlp2_weights, mlp2_bias.reshape(1, -1),
        mlp3_weights, mlp3_bias.reshape(1, -1),
        predict_weights[:, :D], predict_weights[:, D:],
        predict_bias.reshape(1, 1))
    return out
